# baseline (speedup 1.0000x reference)
import functools

import jax
import jax.numpy as jnp
from jax import lax
from jax.experimental import pallas as pl
from jax.experimental.pallas import tpu as pltpu
from jax.experimental.pallas import tpu_sc as plsc

_NT = 16
_K = 80
_NBUF = 4
_SB = 32


def _bcast16(k):
    return jnp.zeros((16,), jnp.int32) + k


_GTR_DNUMS = lax.GatherDimensionNumbers(
    offset_dims=(), collapsed_slice_dims=(0,), start_index_map=(0,))


def _lane_bcast(v16, lane):
    idx = _bcast16(lane).reshape(16, 1)
    return lax.gather(v16, idx, _GTR_DNUMS, slice_sizes=(1,),
                      mode=lax.GatherScatterMode.PROMISE_IN_BOUNDS)


def _sc_pass(eset, si, src_rows, out_hbm, sc, tid, row_lo, cpt,
             take_abs, n, d):
    srcS, dstS, eaS = eset
    acc, src_v, dst_v, ea_v, rows, gsem, ssem = sc
    nq = d // 16
    stripe = (n // (8 * _NT)) * 8
    rem = n - stripe * _NT
    zr = rows[0].shape[0]
    _zero_fill(rows[0], d)
    row0 = pl.multiple_of(tid * stripe, 8)
    for j in range(stripe // zr):
        pltpu.sync_copy(rows[0], acc.at[pl.ds(row0 + j * zr, zr)])
    zrem = stripe - (stripe // zr) * zr
    if zrem:
        pltpu.sync_copy(rows[0].at[pl.ds(0, zrem)],
                        acc.at[pl.ds(row0 + stripe - zrem, zrem)])
    if rem:
        @pl.when(tid == _NT - 1)
        def _():
            pltpu.sync_copy(rows[0].at[pl.ds(0, rem)],
                            acc.at[pl.ds(stripe * _NT, rem)])
    base = pl.multiple_of(row_lo + tid * cpt, 8)
    plsc.subcore_barrier()

    def g_view(j, b):
        return (src_rows.at[src_v.at[pl.ds(j * _K, _K)]], rows[b])

    def s_issue(j, b):
        for g in range(_K // 16):
            d16 = dst_v[pl.ds(j * _K + g * 16, 16)]
            pltpu.async_copy(rows[b].at[pl.ds(g * 16, 16)], acc.at[d16],
                             ssem[b], add=True)

    def s_wait(b):
        d16 = dst_v[pl.ds(0, 16)]
        for g in range(_K // 16):
            pltpu.make_async_copy(rows[b].at[pl.ds(g * 16, 16)],
                                  acc.at[d16], ssem[b]).wait()

    def scale(j, b):
        buf = rows[b]

        def e4(g4, c):
            for u in range(4):
                k = g4 * 4 + u
                kb = lax.shift_left(lax.shift_right_logical(k, 4), 4)
                ea16 = ea_v[pl.ds(j * _K + kb, 16)]
                eab = _lane_bcast(ea16, k - kb)
                for q in range(nq):
                    sl = pl.ds(q * 16, 16)
                    r = buf[k, sl]
                    if take_abs:
                        r = jnp.abs(r)
                    buf[k, sl] = r * eab
            return c

        lax.fori_loop(0, _K // 4, e4, 0)

    def step(j, b):
        bn = (b + 2) % _NBUF
        jn = j + 2

        pltpu.make_async_copy(*g_view(j, b), gsem[b]).wait()
        scale(j, b)
        s_issue(j, b)

        @pl.when(jn < _SB)
        def _():
            @pl.when(j >= _NBUF - 2)
            def _():
                s_wait(bn)
            pltpu.async_copy(*g_view(jn, bn), gsem[bn])

    def rnd(r, c):
        rb = pl.multiple_of(base + r * _SB, 8)
        pltpu.sync_copy(srcS.at[si, pl.ds(rb * _K, _SB * _K)], src_v)
        pltpu.sync_copy(dstS.at[si, pl.ds(rb * _K, _SB * _K)], dst_v)
        pltpu.sync_copy(eaS.at[si, pl.ds(rb * _K, _SB * _K)], ea_v)
        for b in range(2):
            pltpu.async_copy(*g_view(b, b), gsem[b])

        def block(j2, c2):
            for b in range(_NBUF):
                step(j2 * _NBUF + b, b)
            return c2

        lax.fori_loop(0, _SB // _NBUF, block, 0)
        for b in range(_NBUF):
            s_wait(b)
        return c

    lax.fori_loop(0, cpt // _SB, rnd, 0)
    plsc.subcore_barrier()
    pltpu.sync_copy(acc.at[pl.ds(row0, stripe)],
                    out_hbm.at[pl.ds(row0, stripe)])
    if rem:
        @pl.when(tid == _NT - 1)
        def _():
            pltpu.sync_copy(acc.at[pl.ds(stripe * _NT, rem)],
                            out_hbm.at[pl.ds(stripe * _NT, rem)])


def _zero_fill(zero_v, d):
    z = jnp.zeros((16,), jnp.float32)

    def row(i, c):
        for q in range(d // 16):
            zero_v[i, pl.ds(q * 16, 16)] = z
        return c

    lax.fori_loop(0, zero_v.shape[0], row, 0)


def _edge_geom(e):
    rows = e // _K
    half = rows // 2
    hpad = (-half) % (8 * _NT)
    ph = half + hpad
    return rows, half, hpad, ph


def _sc_scratch(n, d, maxc):
    return [
        pltpu.VMEM_SHARED((n, d), jnp.float32),
        pltpu.VMEM((_SB * _K,), jnp.int32),
        pltpu.VMEM((_SB * _K,), jnp.int32),
        pltpu.VMEM((_SB * _K,), jnp.float32),
        [pltpu.VMEM((_K, d), jnp.float32)] * _NBUF,
        [pltpu.SemaphoreType.DMA] * _NBUF,
        [pltpu.SemaphoreType.DMA] * _NBUF,
    ]


def _make_stage1(n, e, d):
    mesh = plsc.VectorSubcoreMesh(core_axis_name="c", subcore_axis_name="s")
    _, _, _, ph = _edge_geom(e)
    cpt = 2 * ph // _NT

    @functools.partial(
        pl.kernel,
        out_type=[jax.ShapeDtypeStruct((n, d), jnp.float32)] * 4,
        mesh=mesh,
        scratch_types=_sc_scratch(n, d, cpt),
    )
    def stage1(x, srcS, dstS, eaS,
               o0, o1, o2, o3, acc, src_v, dst_v, ea_v, rows, gsem, ssem):
        c = lax.axis_index("c")
        t = lax.axis_index("s")
        sc = (acc, src_v, dst_v, ea_v, rows, gsem, ssem)
        es = (srcS, dstS, eaS)

        @pl.when(c == 0)
        def _():
            _sc_pass(es, 0, x, o0, sc, t, 0, cpt, False, n, d)
            _sc_pass(es, 1, x, o1, sc, t, 0, cpt, False, n, d)

        @pl.when(c == 1)
        def _():
            _sc_pass(es, 2, x, o2, sc, t, 0, cpt, False, n, d)
            _sc_pass(es, 3, x, o3, sc, t, 0, cpt, False, n, d)

    return stage1


def _make_stage2(n, e, d):
    mesh = plsc.VectorSubcoreMesh(core_axis_name="c", subcore_axis_name="s")
    _, _, _, ph = _edge_geom(e)
    cpt = 2 * ph // _NT
    hct = ph // _NT

    @functools.partial(
        pl.kernel,
        out_type=[jax.ShapeDtypeStruct((n, d), jnp.float32)] * 4,
        mesh=mesh,
        scratch_types=_sc_scratch(n, d, cpt),
    )
    def stage2(srcS, dstS, eaS, s1, s2, s3,
               x2o, x3o, p0o, p1o, acc, src_v, dst_v, ea_v, rows, gsem,
               ssem):
        c = lax.axis_index("c")
        t = lax.axis_index("s")
        sc = (acc, src_v, dst_v, ea_v, rows, gsem, ssem)
        es = (srcS, dstS, eaS)

        @pl.when(c == 0)
        def _():
            _sc_pass(es, 0, s2, x2o, sc, t, 0, cpt, True, n, d)
            _sc_pass(es, 0, s1, p0o, sc, t, 0, hct, True, n, d)

        @pl.when(c == 1)
        def _():
            _sc_pass(es, 0, s3, x3o, sc, t, 0, cpt, True, n, d)
            _sc_pass(es, 0, s1, p1o, sc, t, ph, hct, True, n, d)

    return stage2


def _elu(h):
    return jnp.where(h > 0, h, jnp.exp(jnp.minimum(h, 0.0)) - 1.0)


def _bn0(h, g, b):
    m = jnp.mean(h, axis=0, keepdims=True)
    v = jnp.mean((h - m) * (h - m), axis=0, keepdims=True)
    return (h - m) * lax.rsqrt(v + 1e-5) * g + b


def _make_dense(n, d, g_seg, c_out):
    def body(x, s0, p0, p1, x2, x3, bat,
             w0, b0, g0, e0, w1, b1, g1, e1, w2, b2, g2, e2, w3, b3, g3, e3,
             f0, fb0, fg0, fe0, f1, fb1, fg1, fe1, f2, fb2, fg2, fe2, out):
        acc = x[:, :]
        branches = (
            (s0[:, :], w0, b0, g0, e0),
            (p0[:, :] + p1[:, :], w1, b1, g1, e1),
            (x2[:, :], w2, b2, g2, e2),
            (x3[:, :], w3, b3, g3, e3),
        )
        for h, w, b, gg, be in branches:
            hh = jnp.dot(_elu(h), w[:, :],
                         preferred_element_type=jnp.float32) + b[:, :]
            acc = acc + _bn0(hh, gg[:, :], be[:, :])
        seg = lax.broadcasted_iota(jnp.int32, (g_seg, n), 0)
        msk = (seg == bat[:, :]).astype(jnp.float32)
        h = jnp.dot(msk, acc, precision=lax.Precision.HIGHEST,
                    preferred_element_type=jnp.float32)
        for j, (fw, fb, fg, fe) in enumerate(
                ((f0, fb0, fg0, fe0), (f1, fb1, fg1, fe1), (f2, fb2, fg2, fe2))):
            h = jnp.dot(h, fw[:, :],
                        preferred_element_type=jnp.float32) + fb[:, :]
            h = _bn0(h, fg[:, :], fe[:, :])
            if j < 2:
                h = jnp.maximum(h, 0.0)
        out[:, :] = h

    return pl.pallas_call(
        body,
        out_shape=jax.ShapeDtypeStruct((g_seg, c_out), jnp.float32),
    )


def _prep_edges(eis, eas, n, e):
    rows, half, hpad, ph = _edge_geom(e)
    he = half * _K
    pe = hpad * _K
    idx = jnp.stack([ei[k] for k in (0, 1) for ei in eis]).reshape(8, 2, he)
    ifill = jnp.broadcast_to(
        (jnp.arange(pe, dtype=jnp.int32) % n)[None, None], (8, 2, pe))
    idx = jnp.concatenate([idx, ifill], axis=2).reshape(8, 2 * ph * _K)
    srcS = idx[:4]
    dstS = idx[4:]
    a = jnp.stack(eas).reshape(4, 2, he)
    zfill = jnp.zeros((4, 2, pe), jnp.float32)
    eaS = jnp.concatenate([a, zfill], axis=2).reshape(4, 2 * ph * _K)
    return srcS, dstS, eaS


def kernel(x, ei0, ei1, ei2, ei3, ea0, ea1, ea2, ea3, batch,
           W0, b0, g0, be0, W1, b1, g1, be1, W2, b2, g2, be2, W3, b3, g3, be3,
           fW0, fb0, fg0, fbe0, fW1, fb1, fg1, fbe1, fW2, fb2, fg2, fbe2):
    n, d = x.shape
    e = ea0.shape[0]
    g_seg = 16
    c_out = fW2.shape[0]

    srcS, dstS, eaS = _prep_edges((ei0, ei1, ei2, ei3),
                                  (ea0, ea1, ea2, ea3), n, e)

    a0, a1, a2, a3 = _make_stage1(n, e, d)(x, srcS, dstS, eaS)
    x2v, x3v, p0v, p1v = _make_stage2(n, e, d)(srcS, dstS, eaS, a1, a2, a3)

    r2 = lambda v: v.reshape(1, -1)
    out = _make_dense(n, d, g_seg, c_out)(
        x, a0, p0v, p1v, x2v, x3v, batch.reshape(1, n),
        W0.T, r2(b0), r2(g0), r2(be0),
        W1.T, r2(b1), r2(g1), r2(be1),
        W2.T, r2(b2), r2(g2), r2(be2),
        W3.T, r2(b3), r2(g3), r2(be3),
        fW0.T, r2(fb0), r2(fg0), r2(fbe0),
        fW1.T, r2(fb1), r2(fg1), r2(fbe1),
        fW2.T, r2(fb2), r2(fg2), r2(fbe2))
    return out


# speedup vs baseline: 1.0999x; 1.0999x over previous
import functools

import jax
import jax.numpy as jnp
from jax import lax
from jax.experimental import pallas as pl
from jax.experimental.pallas import tpu as pltpu
from jax.experimental.pallas import tpu_sc as plsc

_NT = 16
_K = 80
_NBUF = 4
_SB = 32


def _bcast16(k):
    return jnp.zeros((16,), jnp.int32) + k


_GTR_DNUMS = lax.GatherDimensionNumbers(
    offset_dims=(), collapsed_slice_dims=(0,), start_index_map=(0,))


def _lane_bcast(v16, lane):
    idx = _bcast16(lane).reshape(16, 1)
    return lax.gather(v16, idx, _GTR_DNUMS, slice_sizes=(1,),
                      mode=lax.GatherScatterMode.PROMISE_IN_BOUNDS)


def _sc_pass(eset, si, src_rows, out_hbm, sc, tid, row_lo, cpt,
             take_abs, n, d):
    srcS, dstS, eaS = eset
    acc, src_v, dst_v, ea_v, rows, gsem, ssem = sc
    nq = d // 16
    stripe = (n // (8 * _NT)) * 8
    rem = n - stripe * _NT
    zr = rows[0].shape[0]
    _zero_fill(rows[0], d)
    row0 = pl.multiple_of(tid * stripe, 8)
    for j in range(stripe // zr):
        pltpu.sync_copy(rows[0], acc.at[pl.ds(row0 + j * zr, zr)])
    zrem = stripe - (stripe // zr) * zr
    if zrem:
        pltpu.sync_copy(rows[0].at[pl.ds(0, zrem)],
                        acc.at[pl.ds(row0 + stripe - zrem, zrem)])
    if rem:
        @pl.when(tid == _NT - 1)
        def _():
            pltpu.sync_copy(rows[0].at[pl.ds(0, rem)],
                            acc.at[pl.ds(stripe * _NT, rem)])
    base = pl.multiple_of(row_lo + tid * cpt, 8)
    plsc.subcore_barrier()

    def g_view(j, b):
        return (src_rows.at[src_v.at[pl.ds(j * _K, _K)]], rows[b])

    def s_issue(j, b):
        for g in range(_K // 16):
            d16 = dst_v[pl.ds(j * _K + g * 16, 16)]
            pltpu.async_copy(rows[b].at[pl.ds(g * 16, 16)], acc.at[d16],
                             ssem[b], add=True)

    def s_wait(b):
        d16 = dst_v[pl.ds(0, 16)]
        for g in range(_K // 16):
            pltpu.make_async_copy(rows[b].at[pl.ds(g * 16, 16)],
                                  acc.at[d16], ssem[b]).wait()

    def scale(j, b):
        buf = rows[b]

        def e4(g4, c):
            for u in range(4):
                k = g4 * 4 + u
                kb = lax.shift_left(lax.shift_right_logical(k, 4), 4)
                ea16 = ea_v[pl.ds(j * _K + kb, 16)]
                eab = _lane_bcast(ea16, k - kb)
                for q in range(nq):
                    sl = pl.ds(q * 16, 16)
                    r = buf[k, sl]
                    if take_abs:
                        r = jnp.abs(r)
                    buf[k, sl] = r * eab
            return c

        lax.fori_loop(0, _K // 4, e4, 0)

    def step(j, b):
        bn = (b + 2) % _NBUF
        jn = j + 2

        @pl.when(jn < _SB)
        def _():
            @pl.when(j >= _NBUF - 2)
            def _():
                s_wait(bn)
            pltpu.async_copy(*g_view(jn, bn), gsem[bn])

        pltpu.make_async_copy(*g_view(j, b), gsem[b]).wait()
        scale(j, b)
        s_issue(j, b)

    def rnd(r, c):
        rb = pl.multiple_of(base + r * _SB, 8)
        pltpu.sync_copy(srcS.at[si, pl.ds(rb * _K, _SB * _K)], src_v)
        pltpu.sync_copy(dstS.at[si, pl.ds(rb * _K, _SB * _K)], dst_v)
        pltpu.sync_copy(eaS.at[si, pl.ds(rb * _K, _SB * _K)], ea_v)
        for b in range(2):
            pltpu.async_copy(*g_view(b, b), gsem[b])

        def block(j2, c2):
            for b in range(_NBUF):
                step(j2 * _NBUF + b, b)
            return c2

        lax.fori_loop(0, _SB // _NBUF, block, 0)
        for b in range(_NBUF):
            s_wait(b)
        return c

    lax.fori_loop(0, cpt // _SB, rnd, 0)
    plsc.subcore_barrier()
    pltpu.sync_copy(acc.at[pl.ds(row0, stripe)],
                    out_hbm.at[pl.ds(row0, stripe)])
    if rem:
        @pl.when(tid == _NT - 1)
        def _():
            pltpu.sync_copy(acc.at[pl.ds(stripe * _NT, rem)],
                            out_hbm.at[pl.ds(stripe * _NT, rem)])


def _zero_fill(zero_v, d):
    z = jnp.zeros((16,), jnp.float32)

    def row(i, c):
        for q in range(d // 16):
            zero_v[i, pl.ds(q * 16, 16)] = z
        return c

    lax.fori_loop(0, zero_v.shape[0], row, 0)


def _edge_geom(e):
    rows = e // _K
    half = rows // 2
    hpad = (-half) % (8 * _NT)
    ph = half + hpad
    return rows, half, hpad, ph


def _sc_scratch(n, d, maxc):
    return [
        pltpu.VMEM_SHARED((n, d), jnp.float32),
        pltpu.VMEM((_SB * _K,), jnp.int32),
        pltpu.VMEM((_SB * _K,), jnp.int32),
        pltpu.VMEM((_SB * _K,), jnp.float32),
        [pltpu.VMEM((_K, d), jnp.float32)] * _NBUF,
        [pltpu.SemaphoreType.DMA] * _NBUF,
        [pltpu.SemaphoreType.DMA] * _NBUF,
    ]


def _make_stage1(n, e, d):
    mesh = plsc.VectorSubcoreMesh(core_axis_name="c", subcore_axis_name="s")
    _, _, _, ph = _edge_geom(e)
    cpt = 2 * ph // _NT

    @functools.partial(
        pl.kernel,
        out_type=[jax.ShapeDtypeStruct((n, d), jnp.float32)] * 4,
        mesh=mesh,
        scratch_types=_sc_scratch(n, d, cpt),
    )
    def stage1(x, srcS, dstS, eaS,
               o0, o1, o2, o3, acc, src_v, dst_v, ea_v, rows, gsem, ssem):
        c = lax.axis_index("c")
        t = lax.axis_index("s")
        sc = (acc, src_v, dst_v, ea_v, rows, gsem, ssem)
        es = (srcS, dstS, eaS)

        @pl.when(c == 0)
        def _():
            _sc_pass(es, 0, x, o0, sc, t, 0, cpt, False, n, d)
            _sc_pass(es, 1, x, o1, sc, t, 0, cpt, False, n, d)

        @pl.when(c == 1)
        def _():
            _sc_pass(es, 2, x, o2, sc, t, 0, cpt, False, n, d)
            _sc_pass(es, 3, x, o3, sc, t, 0, cpt, False, n, d)

    return stage1


def _make_stage2(n, e, d):
    mesh = plsc.VectorSubcoreMesh(core_axis_name="c", subcore_axis_name="s")
    _, _, _, ph = _edge_geom(e)
    cpt = 2 * ph // _NT
    hct = ph // _NT

    @functools.partial(
        pl.kernel,
        out_type=[jax.ShapeDtypeStruct((n, d), jnp.float32)] * 4,
        mesh=mesh,
        scratch_types=_sc_scratch(n, d, cpt),
    )
    def stage2(srcS, dstS, eaS, s1, s2, s3,
               x2o, x3o, p0o, p1o, acc, src_v, dst_v, ea_v, rows, gsem,
               ssem):
        c = lax.axis_index("c")
        t = lax.axis_index("s")
        sc = (acc, src_v, dst_v, ea_v, rows, gsem, ssem)
        es = (srcS, dstS, eaS)

        @pl.when(c == 0)
        def _():
            _sc_pass(es, 0, s2, x2o, sc, t, 0, cpt, True, n, d)
            _sc_pass(es, 0, s1, p0o, sc, t, 0, hct, True, n, d)

        @pl.when(c == 1)
        def _():
            _sc_pass(es, 0, s3, x3o, sc, t, 0, cpt, True, n, d)
            _sc_pass(es, 0, s1, p1o, sc, t, ph, hct, True, n, d)

    return stage2


def _elu(h):
    return jnp.where(h > 0, h, jnp.exp(jnp.minimum(h, 0.0)) - 1.0)


def _bn0(h, g, b):
    m = jnp.mean(h, axis=0, keepdims=True)
    v = jnp.mean((h - m) * (h - m), axis=0, keepdims=True)
    return (h - m) * lax.rsqrt(v + 1e-5) * g + b


def _make_dense(n, d, g_seg, c_out):
    def body(x, s0, p0, p1, x2, x3, bat,
             w0, b0, g0, e0, w1, b1, g1, e1, w2, b2, g2, e2, w3, b3, g3, e3,
             f0, fb0, fg0, fe0, f1, fb1, fg1, fe1, f2, fb2, fg2, fe2, out):
        acc = x[:, :]
        branches = (
            (s0[:, :], w0, b0, g0, e0),
            (p0[:, :] + p1[:, :], w1, b1, g1, e1),
            (x2[:, :], w2, b2, g2, e2),
            (x3[:, :], w3, b3, g3, e3),
        )
        for h, w, b, gg, be in branches:
            hh = jnp.dot(_elu(h), w[:, :],
                         preferred_element_type=jnp.float32) + b[:, :]
            acc = acc + _bn0(hh, gg[:, :], be[:, :])
        seg = lax.broadcasted_iota(jnp.int32, (g_seg, n), 0)
        msk = (seg == bat[:, :]).astype(jnp.float32)
        h = jnp.dot(msk, acc, precision=lax.Precision.HIGHEST,
                    preferred_element_type=jnp.float32)
        for j, (fw, fb, fg, fe) in enumerate(
                ((f0, fb0, fg0, fe0), (f1, fb1, fg1, fe1), (f2, fb2, fg2, fe2))):
            h = jnp.dot(h, fw[:, :],
                        preferred_element_type=jnp.float32) + fb[:, :]
            h = _bn0(h, fg[:, :], fe[:, :])
            if j < 2:
                h = jnp.maximum(h, 0.0)
        out[:, :] = h

    return pl.pallas_call(
        body,
        out_shape=jax.ShapeDtypeStruct((g_seg, c_out), jnp.float32),
    )


def _prep_edges(eis, eas, n, e):
    rows, half, hpad, ph = _edge_geom(e)
    he = half * _K
    pe = hpad * _K
    idx = jnp.stack([ei[k] for k in (0, 1) for ei in eis]).reshape(8, 2, he)
    ifill = jnp.broadcast_to(
        (jnp.arange(pe, dtype=jnp.int32) % n)[None, None], (8, 2, pe))
    idx = jnp.concatenate([idx, ifill], axis=2).reshape(8, 2 * ph * _K)
    srcS = idx[:4]
    dstS = idx[4:]
    a = jnp.stack(eas).reshape(4, 2, he)
    zfill = jnp.zeros((4, 2, pe), jnp.float32)
    eaS = jnp.concatenate([a, zfill], axis=2).reshape(4, 2 * ph * _K)
    return srcS, dstS, eaS


def kernel(x, ei0, ei1, ei2, ei3, ea0, ea1, ea2, ea3, batch,
           W0, b0, g0, be0, W1, b1, g1, be1, W2, b2, g2, be2, W3, b3, g3, be3,
           fW0, fb0, fg0, fbe0, fW1, fb1, fg1, fbe1, fW2, fb2, fg2, fbe2):
    n, d = x.shape
    e = ea0.shape[0]
    g_seg = 16
    c_out = fW2.shape[0]

    srcS, dstS, eaS = _prep_edges((ei0, ei1, ei2, ei3),
                                  (ea0, ea1, ea2, ea3), n, e)

    a0, a1, a2, a3 = _make_stage1(n, e, d)(x, srcS, dstS, eaS)
    x2v, x3v, p0v, p1v = _make_stage2(n, e, d)(srcS, dstS, eaS, a1, a2, a3)

    r2 = lambda v: v.reshape(1, -1)
    out = _make_dense(n, d, g_seg, c_out)(
        x, a0, p0v, p1v, x2v, x3v, batch.reshape(1, n),
        W0.T, r2(b0), r2(g0), r2(be0),
        W1.T, r2(b1), r2(g1), r2(be1),
        W2.T, r2(b2), r2(g2), r2(be2),
        W3.T, r2(b3), r2(g3), r2(be3),
        fW0.T, r2(fb0), r2(fg0), r2(fbe0),
        fW1.T, r2(fb1), r2(fg1), r2(fbe1),
        fW2.T, r2(fb2), r2(fg2), r2(fbe2))
    return out
